# Initial kernel scaffold; baseline (speedup 1.0000x reference)
import jax
import jax.numpy as jnp
from jax.experimental import pallas as pl


def kernel(embeds, centroids):
    raise NotImplementedError("write your pallas kernel here")


# speedup vs baseline: 32.9121x; 32.9121x over previous
import functools

import jax
import jax.numpy as jnp
from jax.experimental import pallas as pl
from jax.experimental.pallas import tpu as pltpu

_NBLK = 512


def _knn_softmin_kernel(e_ref, c_ref, out_ref):
    e = e_ref[...]
    c = c_ref[...]

    feat = jnp.sum(e * e, axis=1, keepdims=True)
    ones = jnp.ones((1, c.shape[1]), dtype=jnp.float32)
    cent = jax.lax.dot_general(
        ones, c * c,
        dimension_numbers=(((1,), (1,)), ((), ())),
        preferred_element_type=jnp.float32,
    )

    f_c = 2.0 * jax.lax.dot_general(
        e, c,
        dimension_numbers=(((1,), (1,)), ((), ())),
        preferred_element_type=jnp.float32,
    )
    d2 = feat + cent - f_c

    lane = jax.lax.broadcasted_iota(jnp.int32, d2.shape, 1)
    big = jnp.float32(jnp.inf)

    m0 = jnp.min(d2, axis=1, keepdims=True)
    i0 = jnp.min(jnp.where(d2 == m0, lane, d2.shape[1]), axis=1, keepdims=True)
    d2m = jnp.where(lane == i0, big, d2)

    m1 = jnp.min(d2m, axis=1, keepdims=True)
    i1 = jnp.min(jnp.where(d2m == m1, lane, d2.shape[1]), axis=1, keepdims=True)
    d2m = jnp.where(lane == i1, big, d2m)

    m2 = jnp.min(d2m, axis=1, keepdims=True)

    d0 = jnp.sqrt(m0)
    d1 = jnp.sqrt(m1)
    d2v = jnp.sqrt(m2)

    s0 = 1.0 / (1.0 + jnp.exp(d0 - d1) + jnp.exp(d0 - d2v))
    out_ref[...] = s0 * d0


@jax.jit
def kernel(embeds, centroids):
    b, n, d = embeds.shape
    p = centroids.shape[0]
    bn = b * n
    e2 = embeds.reshape(bn, d)

    grid = (bn // _NBLK,)
    out = pl.pallas_call(
        _knn_softmin_kernel,
        grid=grid,
        in_specs=[
            pl.BlockSpec((_NBLK, d), lambda i: (i, 0)),
            pl.BlockSpec((p, d), lambda i: (0, 0)),
        ],
        out_specs=pl.BlockSpec((_NBLK, 1), lambda i: (i, 0)),
        out_shape=jax.ShapeDtypeStruct((bn, 1), jnp.float32),
        compiler_params=pltpu.CompilerParams(
            dimension_semantics=("arbitrary",),
        ),
    )(e2, centroids)

    h = 32
    w = n // h
    return out.reshape(b, h, w, 1).transpose(0, 3, 1, 2)


# speedup vs baseline: 39.9931x; 1.2151x over previous
import functools

import jax
import jax.numpy as jnp
from jax.experimental import pallas as pl
from jax.experimental.pallas import tpu as pltpu

_NBLK = 512
_NCHUNK = 4


def _merge3(x, y):
    x0, x1, x2 = x
    y0, y1, y2 = y
    z0 = jnp.minimum(x0, y0)
    z1 = jnp.minimum(jnp.maximum(x0, y0), jnp.minimum(x1, y1))
    z2 = jnp.minimum(jnp.minimum(x2, y2),
                     jnp.minimum(jnp.maximum(x1, y0), jnp.maximum(x0, y1)))
    return z0, z1, z2


def _chunk_top3(d2):
    n, c = d2.shape
    h = c // 2
    x, y = d2[:, :h], d2[:, h:]
    t0 = jnp.minimum(x, y)
    t1 = jnp.maximum(x, y)
    q = h // 2
    x0, x1 = t0[:, :q], t1[:, :q]
    y0, y1 = t0[:, q:], t1[:, q:]
    z0 = jnp.minimum(x0, y0)
    z1 = jnp.minimum(jnp.maximum(x0, y0), jnp.minimum(x1, y1))
    z2 = jnp.minimum(jnp.maximum(x1, y0), jnp.maximum(x0, y1))
    return z0, z1, z2


def _knn_softmin_kernel(e_ref, c_ref, out_ref):
    e = e_ref[...]
    feat = jnp.sum(e * e, axis=1, keepdims=True)

    p, d = c_ref.shape
    ch = p // _NCHUNK
    ones = jnp.ones((1, d), dtype=jnp.float32)
    acc = None
    for k in range(_NCHUNK):
        c = c_ref[k * ch:(k + 1) * ch, :]
        cent = jax.lax.dot_general(
            ones, c * c,
            dimension_numbers=(((1,), (1,)), ((), ())),
            preferred_element_type=jnp.float32,
        )
        f_c = 2.0 * jax.lax.dot_general(
            e, c,
            dimension_numbers=(((1,), (1,)), ((), ())),
            preferred_element_type=jnp.float32,
        )
        d2 = feat + cent - f_c
        tri = _chunk_top3(d2)
        acc = tri if acc is None else _merge3(acc, tri)

    cand = jnp.concatenate(acc, axis=1)
    w = cand.shape[1]
    lane = jax.lax.broadcasted_iota(jnp.int32, cand.shape, 1)
    big = jnp.float32(jnp.inf)

    m0 = jnp.min(cand, axis=1, keepdims=True)
    i0 = jnp.min(jnp.where(cand == m0, lane, w), axis=1, keepdims=True)
    cand = jnp.where(lane == i0, big, cand)

    m1 = jnp.min(cand, axis=1, keepdims=True)
    i1 = jnp.min(jnp.where(cand == m1, lane, w), axis=1, keepdims=True)
    cand = jnp.where(lane == i1, big, cand)

    m2 = jnp.min(cand, axis=1, keepdims=True)

    d0 = jnp.sqrt(m0)
    d1 = jnp.sqrt(m1)
    d2v = jnp.sqrt(m2)

    s0 = 1.0 / (1.0 + jnp.exp(d0 - d1) + jnp.exp(d0 - d2v))
    out_ref[...] = s0 * d0


@jax.jit
def kernel(embeds, centroids):
    b, n, d = embeds.shape
    p = centroids.shape[0]
    bn = b * n
    e2 = embeds.reshape(bn, d)

    grid = (bn // _NBLK,)
    out = pl.pallas_call(
        _knn_softmin_kernel,
        grid=grid,
        in_specs=[
            pl.BlockSpec((_NBLK, d), lambda i: (i, 0)),
            pl.BlockSpec((p, d), lambda i: (0, 0)),
        ],
        out_specs=pl.BlockSpec((_NBLK, 1), lambda i: (i, 0)),
        out_shape=jax.ShapeDtypeStruct((bn, 1), jnp.float32),
        compiler_params=pltpu.CompilerParams(
            dimension_semantics=("arbitrary",),
        ),
    )(e2, centroids)

    h = 32
    w = n // h
    return out.reshape(b, h, w, 1).transpose(0, 3, 1, 2)


# speedup vs baseline: 51.6558x; 1.2916x over previous
import functools

import jax
import jax.numpy as jnp
from jax.experimental import pallas as pl
from jax.experimental.pallas import tpu as pltpu

_NBLK = 512
_NCHUNK = 4


def _merge3(x, y):
    x0, x1, x2 = x
    y0, y1, y2 = y
    z0 = jnp.minimum(x0, y0)
    z1 = jnp.minimum(jnp.maximum(x0, y0), jnp.minimum(x1, y1))
    z2 = jnp.minimum(jnp.minimum(x2, y2),
                     jnp.minimum(jnp.maximum(x1, y0), jnp.maximum(x0, y1)))
    return z0, z1, z2


def _chunk_top3(d2):
    n, c = d2.shape
    h = c // 2
    x, y = d2[:, :h], d2[:, h:]
    t0 = jnp.minimum(x, y)
    t1 = jnp.maximum(x, y)
    q = h // 2
    x0, x1 = t0[:, :q], t1[:, :q]
    y0, y1 = t0[:, q:], t1[:, q:]
    z0 = jnp.minimum(x0, y0)
    z1 = jnp.minimum(jnp.maximum(x0, y0), jnp.minimum(x1, y1))
    z2 = jnp.minimum(jnp.maximum(x1, y0), jnp.maximum(x0, y1))
    return z0, z1, z2


def _knn_softmin_kernel(e_ref, c_ref, out_ref, cent_ref):
    p, d = c_ref.shape
    ch = p // _NCHUNK

    @pl.when(pl.program_id(0) == 0)
    def _init_cent():
        ones_half = jnp.full((1, d), 0.5, dtype=jnp.float32)
        for k in range(_NCHUNK):
            c = c_ref[k * ch:(k + 1) * ch, :]
            cent_ref[:, k * ch:(k + 1) * ch] = jax.lax.dot_general(
                ones_half, c * c,
                dimension_numbers=(((1,), (1,)), ((), ())),
                preferred_element_type=jnp.float32,
            )

    e = e_ref[...]
    feat = jnp.sum(e * e, axis=1, keepdims=True)

    acc = None
    for k in range(_NCHUNK):
        c = c_ref[k * ch:(k + 1) * ch, :]
        f = jax.lax.dot_general(
            e, c,
            dimension_numbers=(((1,), (1,)), ((), ())),
            preferred_element_type=jnp.float32,
        )
        s = cent_ref[:, k * ch:(k + 1) * ch] - f
        tri = _chunk_top3(s)
        acc = tri if acc is None else _merge3(acc, tri)

    cand = jnp.concatenate(acc, axis=1)
    w = cand.shape[1]
    lane = jax.lax.broadcasted_iota(jnp.int32, cand.shape, 1)
    big = jnp.float32(jnp.inf)

    m0 = jnp.min(cand, axis=1, keepdims=True)
    i0 = jnp.min(jnp.where(cand == m0, lane, w), axis=1, keepdims=True)
    cand = jnp.where(lane == i0, big, cand)

    m1 = jnp.min(cand, axis=1, keepdims=True)
    i1 = jnp.min(jnp.where(cand == m1, lane, w), axis=1, keepdims=True)
    cand = jnp.where(lane == i1, big, cand)

    m2 = jnp.min(cand, axis=1, keepdims=True)

    d0 = jnp.sqrt(2.0 * m0 + feat)
    d1 = jnp.sqrt(2.0 * m1 + feat)
    d2v = jnp.sqrt(2.0 * m2 + feat)

    s0 = 1.0 / (1.0 + jnp.exp(d0 - d1) + jnp.exp(d0 - d2v))
    out_ref[...] = s0 * d0


@jax.jit
def kernel(embeds, centroids):
    b, n, d = embeds.shape
    p = centroids.shape[0]
    bn = b * n
    e2 = embeds.reshape(bn, d)

    grid = (bn // _NBLK,)
    out = pl.pallas_call(
        _knn_softmin_kernel,
        grid=grid,
        in_specs=[
            pl.BlockSpec((_NBLK, d), lambda i: (i, 0)),
            pl.BlockSpec((p, d), lambda i: (0, 0)),
        ],
        out_specs=pl.BlockSpec((_NBLK, 1), lambda i: (i, 0)),
        out_shape=jax.ShapeDtypeStruct((bn, 1), jnp.float32),
        scratch_shapes=[pltpu.VMEM((1, p), jnp.float32)],
        compiler_params=pltpu.CompilerParams(
            dimension_semantics=("arbitrary",),
        ),
    )(e2, centroids)

    h = 32
    w = n // h
    return out.reshape(b, h, w, 1).transpose(0, 3, 1, 2)
